# baseline (speedup 1.0000x reference)
import functools

import jax
import jax.numpy as jnp
import numpy as np
from jax import lax
from jax.experimental import pallas as pl
from jax.experimental.pallas import tpu as pltpu
from jax.experimental.pallas import tpu_sc as plsc

_ANCHORS = np.array(
    [[[116., 90.], [156., 198.], [373., 326.]],
     [[30., 61.], [62., 45.], [59., 119.]],
     [[10., 13.], [16., 30.], [33., 23.]]], dtype=np.float32)
_GRID_SIZES = (19, 38, 76)
_INP_DIM = 608.0
_NUM_ANCH = 3
_L_COORD = 1.0
_L_OBJ = 5.0
_L_NOOBJ = 0.5
_B, _T = 16, 8
_N = 3 * (19 * 19 + 38 * 38 + 76 * 76)
_C = 85
_NUM_CLASSES = 80
_TOT = _B * _N

_NW = 32
_CH = 128
_NCH = 89
_NBUF = 8
_LN2 = 0.6931471805599453
_SQRTH = 1.4142135


def _poly_neg_clip_log(x):
    xb = lax.bitcast_convert_type(x, jnp.int32)
    e = ((xb >> 23) & 0xFF) - 127
    m = lax.bitcast_convert_type((xb & 0x007FFFFF) | 0x3F800000, jnp.float32)
    big = m > _SQRTH
    m = jnp.where(big, m * 0.5, m)
    e = e + jnp.where(big, jnp.int32(1), jnp.int32(0))
    t = m - 1.0
    y = jnp.float32(7.0376836292e-2)
    for c in (-1.1514610310e-1, 1.1676998740e-1, -1.2420140846e-1,
              1.4249322787e-1, -1.6668057665e-1, 2.0000714765e-1,
              -2.4999993993e-1, 3.3333331174e-1):
        y = y * t + c
    z = t * t
    r = t - 0.5 * z + t * z * y + e.astype(jnp.float32) * _LN2
    r = jnp.where(x > 0.0, r, -100.0)
    return jnp.maximum(r, -100.0)


def _conf_reduce_body(pred_hbm, out_hbm, idx_ref, conf_ref, acc_ref, sems):
    wid = lax.axis_index("s") * 2 + lax.axis_index("c")
    base = wid * (_NCH * _CH)
    iota = lax.iota(jnp.int32, 16)

    def build(g, carry):
        for l in range(_CH // 16):
            pos = base + g * _CH + l * 16 + iota
            n = jnp.minimum(pos, _TOT - 1)
            idx_ref[g, pl.ds(l * 16, 16)] = n * _C + 4
        return carry

    lax.fori_loop(0, _NCH, build, 0)

    def start(g, slot):
        pltpu.make_async_copy(
            pred_hbm.at[idx_ref.at[g]], conf_ref.at[g], sems.at[slot]).start()

    def wait(g, slot):
        pltpu.make_async_copy(
            pred_hbm.at[idx_ref.at[g]], conf_ref.at[g], sems.at[slot]).wait()

    for b in range(_NBUF):
        start(b, b)

    def mbody(g, acc):
        slot = lax.rem(g, _NBUF)
        wait(g, slot)
        gn = jnp.minimum(g + _NBUF, _NCH - 1)

        @pl.when(g + _NBUF < _NCH)
        def _():
            start(gn, slot)

        for l in range(_CH // 16):
            c = conf_ref[g, pl.ds(l * 16, 16)]
            pos = base + g * _CH + l * 16 + iota
            v = _poly_neg_clip_log(1.0 - c)
            acc = acc + jnp.where(pos < _TOT, v, 0.0)
        return acc

    acc = lax.fori_loop(0, _NCH, mbody, jnp.zeros((16,), jnp.float32))
    acc_ref[...] = acc
    pltpu.sync_copy(acc_ref, out_hbm.at[wid])


@jax.jit
def _conf_reduce(pred_flat):
    mesh = plsc.VectorSubcoreMesh(core_axis_name="c", subcore_axis_name="s")
    return pl.kernel(
        _conf_reduce_body,
        out_type=jax.ShapeDtypeStruct((_NW, 16), jnp.float32),
        mesh=mesh,
        scratch_types=[
            pltpu.VMEM((_NCH, _CH), jnp.int32),
            pltpu.VMEM((_NCH, _CH), jnp.float32),
            pltpu.VMEM((16,), jnp.float32),
            pltpu.SemaphoreType.DMA((_NBUF,)),
        ],
    )(pred_flat)


def kernel(pred_x, coord_x, y_cls, y_coord):
    pred_flat = pred_x.reshape(-1)
    l1m_total = jnp.sum(_conf_reduce(pred_flat))

    boxes = y_coord.reshape(-1, 4)
    m_arange = jnp.arange(_B * _T, dtype=jnp.int32)
    rows = jnp.repeat(jnp.arange(_B, dtype=jnp.int32), _T)
    cand_parts = []
    base = 0
    for g in _GRID_SIZES:
        idx0 = base + ((jnp.floor(boxes[:, 1] * g)
                        + jnp.floor(boxes[:, 0] * g) * g) * _NUM_ANCH
                       ).astype(jnp.int32)
        cand_parts.append(idx0[:, None]
                          + jnp.arange(_NUM_ANCH, dtype=jnp.int32)[None, :])
        base += g * g * _NUM_ANCH
    candis = jnp.concatenate(cand_parts, axis=1)

    tb = jnp.stack([_INP_DIM * (boxes[:, 0] - boxes[:, 2] / 2),
                    _INP_DIM * (boxes[:, 1] - boxes[:, 3] / 2),
                    _INP_DIM * (boxes[:, 0] + boxes[:, 2] / 2),
                    _INP_DIM * (boxes[:, 1] + boxes[:, 3] / 2)], axis=1)

    cand_rows = pred_x[rows[:, None], candis]
    cxy, cwh = cand_rows[:, :, :2], cand_rows[:, :, 2:4]
    c1 = cxy - cwh / 2.0
    c2 = cxy + cwh / 2.0
    ix1 = jnp.maximum(tb[:, None, 0], c1[:, :, 0])
    iy1 = jnp.maximum(tb[:, None, 1], c1[:, :, 1])
    ix2 = jnp.minimum(tb[:, None, 2], c2[:, :, 0])
    iy2 = jnp.minimum(tb[:, None, 3], c2[:, :, 1])
    inter = jnp.maximum(ix2 - ix1, 0.0) * jnp.maximum(iy2 - iy1, 0.0)
    a1 = (tb[:, 2] - tb[:, 0]) * (tb[:, 3] - tb[:, 1])
    a2 = (c2[:, :, 0] - c1[:, :, 0]) * (c2[:, :, 1] - c1[:, :, 1])
    ious = inter / (a1[:, None] + a2 - inter + 1e-16)
    ti = jnp.argmax(ious, axis=1).astype(jnp.int32)
    cols = jnp.take_along_axis(candis, ti[:, None], axis=1)[:, 0]
    gidx = ti // _NUM_ANCH
    aidx = ti % _NUM_ANCH

    grid_f = jnp.asarray(np.array(_GRID_SIZES, dtype=np.float32))[gidx]
    anchor = jnp.asarray(_ANCHORS)[gidx, aidx]
    txy = (boxes[:, :2] * grid_f[:, None]) % 1.0 + 1e-05
    txy = jnp.log(txy / (1.0 - txy))
    twh = jnp.log(boxes[:, 2:] * _INP_DIM / anchor)
    target_coord = jnp.concatenate([txy, twh], axis=-1)

    sel = pred_x[rows, cols]
    coord_sel = coord_x[rows, cols]
    coord_loss = _L_COORD * jnp.sum((coord_sel - target_coord) ** 2)

    tcls = jax.nn.one_hot(y_cls.reshape(-1), _NUM_CLASSES, dtype=jnp.float32)
    cls_pred = sel[:, 5:]
    cls_loss = -jnp.sum(
        jnp.clip(jnp.log(cls_pred), -100.0, None) * tcls
        + jnp.clip(jnp.log(1.0 - cls_pred), -100.0, None) * (1.0 - tcls))

    flat_ids = rows * _N + cols
    dup = (flat_ids[:, None] == flat_ids[None, :]) & (
        m_arange[None, :] < m_arange[:, None])
    first = ~jnp.any(dup, axis=1)
    conf_g = sel[:, 4]
    logp = jnp.clip(jnp.log(conf_g), -100.0, None)
    l1m = jnp.clip(jnp.log(1.0 - conf_g), -100.0, None)
    fw = first.astype(jnp.float32)
    conf_loss = (_L_OBJ * (-jnp.sum(logp * fw))
                 + _L_NOOBJ * (-(l1m_total - jnp.sum(l1m * fw))))

    return coord_loss + conf_loss + cls_loss


# speedup vs baseline: 3.6446x; 3.6446x over previous
import functools

import jax
import jax.numpy as jnp
import numpy as np
from jax import lax
from jax.experimental import pallas as pl

_GRID_SIZES = (19, 38, 76)
_INP_DIM = 608.0
_NUM_ANCH = 3
_L_COORD = 1.0
_L_OBJ = 5.0
_L_NOOBJ = 0.5
_B, _T = 16, 8
_N = 3 * (19 * 19 + 38 * 38 + 76 * 76)
_C = 85
_NUM_CLASSES = 80
_M = _B * _T
_K = 9

_TN = 2048
_NSTEP = -(-_N // _TN)

_AW = (116., 156., 373., 30., 62., 59., 10., 16., 33.)
_AH = (90., 198., 326., 61., 45., 119., 13., 30., 23.)


def _a_body(confT_ref, cx_ref, cy_ref, cw_ref, ch_ref, candis_ref, tb_ref,
            noobj_ref, cols_ref, ti_ref):
    j = pl.program_id(0)
    conf = confT_ref[0]
    lane = lax.broadcasted_iota(jnp.int32, (_B, _TN), 1) + j * _TN
    x = jnp.where(lane < _N, 1.0 - conf, 1.0)
    s = jnp.sum(jnp.maximum(jnp.log(x), -100.0))

    @pl.when(j == 0)
    def _init():
        noobj_ref[...] = s.reshape(1, 1)
        cx, cy = cx_ref[...], cy_ref[...]
        cw, ch = cw_ref[...], ch_ref[...]
        x1, y1 = cx - cw * 0.5, cy - ch * 0.5
        x2, y2 = cx + cw * 0.5, cy + ch * 0.5
        tb = tb_ref[...]
        ix1 = jnp.maximum(tb[:, 0:1], x1)
        iy1 = jnp.maximum(tb[:, 1:2], y1)
        ix2 = jnp.minimum(tb[:, 2:3], x2)
        iy2 = jnp.minimum(tb[:, 3:4], y2)
        inter = jnp.maximum(ix2 - ix1, 0.0) * jnp.maximum(iy2 - iy1, 0.0)
        a1 = (tb[:, 2:3] - tb[:, 0:1]) * (tb[:, 3:4] - tb[:, 1:2])
        a2 = (x2 - x1) * (y2 - y1)
        iou = inter / (a1 + a2 - inter + 1e-16)
        kio = lax.broadcasted_iota(jnp.int32, (_M, _K), 1)
        mx = jnp.max(iou, axis=1, keepdims=True)
        ti = jnp.min(jnp.where(iou == mx, kio, _K), axis=1, keepdims=True)
        cols_ref[...] = jnp.sum(
            jnp.where(kio == ti, candis_ref[...], 0), axis=1, keepdims=True)
        ti_ref[...] = ti

    @pl.when(j > 0)
    def _acc():
        noobj_ref[...] += s.reshape(1, 1)


def _b_body(noobj_ref, ids_c_ref, ids_r_ref, ti_ref, cls_ref, conf_ref,
            csel_ref, boxes_ref, ycls_ref, out_ref):
    ti = ti_ref[...]
    gidx = ti // _NUM_ANCH
    gf = jnp.where(gidx == 0, 19.0, jnp.where(gidx == 1, 38.0, 76.0))
    aw = jnp.full((_M, 1), _AW[0], jnp.float32)
    ah = jnp.full((_M, 1), _AH[0], jnp.float32)
    for k in range(1, _K):
        aw = jnp.where(ti == k, _AW[k], aw)
        ah = jnp.where(ti == k, _AH[k], ah)
    boxes = boxes_ref[...]
    bx, by = boxes[:, 0:1], boxes[:, 1:2]
    bw, bh = boxes[:, 2:3], boxes[:, 3:4]
    fx = bx * gf
    fy = by * gf
    fx = fx - jnp.floor(fx) + 1e-05
    fy = fy - jnp.floor(fy) + 1e-05
    tx = jnp.log(fx / (1.0 - fx))
    ty = jnp.log(fy / (1.0 - fy))
    tw = jnp.log(bw * _INP_DIM / aw)
    th = jnp.log(bh * _INP_DIM / ah)
    cs = csel_ref[...]
    coord_loss = _L_COORD * jnp.sum(
        (cs[:, 0:1] - tx) ** 2 + (cs[:, 1:2] - ty) ** 2
        + (cs[:, 2:3] - tw) ** 2 + (cs[:, 3:4] - th) ** 2)

    c80 = lax.broadcasted_iota(jnp.int32, (_M, _NUM_CLASSES), 1)
    tcls = jnp.where(c80 == ycls_ref[...], 1.0, 0.0)
    p = cls_ref[...]
    cls_loss = -jnp.sum(
        jnp.maximum(jnp.log(p), -100.0) * tcls
        + jnp.maximum(jnp.log(1.0 - p), -100.0) * (1.0 - tcls))

    eq = ids_c_ref[...] == ids_r_ref[...]
    lower = (lax.broadcasted_iota(jnp.int32, (_M, _M), 1)
             < lax.broadcasted_iota(jnp.int32, (_M, _M), 0))
    ndup = jnp.sum(jnp.where(eq & lower, 1.0, 0.0), axis=1, keepdims=True)
    first = jnp.where(ndup > 0.0, 0.0, 1.0)
    cg = conf_ref[...]
    logp = jnp.maximum(jnp.log(cg), -100.0)
    l1m = jnp.maximum(jnp.log(1.0 - cg), -100.0)
    conf_loss = (_L_OBJ * (-jnp.sum(logp * first))
                 + _L_NOOBJ * (-(noobj_ref[0, 0] - jnp.sum(l1m * first))))
    out_ref[...] = (coord_loss + conf_loss + cls_loss).reshape(1, 1)


def kernel(pred_x, coord_x, y_cls, y_coord):
    f32 = jnp.float32
    predT = jnp.transpose(pred_x, (2, 0, 1))

    boxes = y_coord.reshape(-1, 4)
    rows = jnp.repeat(jnp.arange(_B, dtype=jnp.int32), _T)
    cand_parts = []
    base = 0
    for g in _GRID_SIZES:
        idx0 = base + ((jnp.floor(boxes[:, 1] * g)
                        + jnp.floor(boxes[:, 0] * g) * g) * _NUM_ANCH
                       ).astype(jnp.int32)
        cand_parts.append(idx0[:, None]
                          + jnp.arange(_NUM_ANCH, dtype=jnp.int32)[None, :])
        base += g * g * _NUM_ANCH
    candis = jnp.concatenate(cand_parts, axis=1)
    tb = jnp.stack([_INP_DIM * (boxes[:, 0] - boxes[:, 2] / 2),
                    _INP_DIM * (boxes[:, 1] - boxes[:, 3] / 2),
                    _INP_DIM * (boxes[:, 0] + boxes[:, 2] / 2),
                    _INP_DIM * (boxes[:, 1] + boxes[:, 3] / 2)], axis=1)

    cand = pred_x[rows[:, None], candis]
    cxc, cyc = cand[..., 0], cand[..., 1]
    cwc, chc = cand[..., 2], cand[..., 3]

    noobj, cols2, ti2 = pl.pallas_call(
        _a_body,
        grid=(_NSTEP,),
        in_specs=[
            pl.BlockSpec((1, _B, _TN), lambda j: (4, 0, j)),
            pl.BlockSpec((_M, _K), lambda j: (0, 0)),
            pl.BlockSpec((_M, _K), lambda j: (0, 0)),
            pl.BlockSpec((_M, _K), lambda j: (0, 0)),
            pl.BlockSpec((_M, _K), lambda j: (0, 0)),
            pl.BlockSpec((_M, _K), lambda j: (0, 0)),
            pl.BlockSpec((_M, 4), lambda j: (0, 0)),
        ],
        out_specs=[
            pl.BlockSpec((1, 1), lambda j: (0, 0)),
            pl.BlockSpec((_M, 1), lambda j: (0, 0)),
            pl.BlockSpec((_M, 1), lambda j: (0, 0)),
        ],
        out_shape=[
            jax.ShapeDtypeStruct((1, 1), f32),
            jax.ShapeDtypeStruct((_M, 1), jnp.int32),
            jax.ShapeDtypeStruct((_M, 1), jnp.int32),
        ],
    )(predT, cxc, cyc, cwc, chc, candis, tb)

    cols = cols2.reshape(-1)
    sel = pred_x[rows, cols]
    csel = coord_x[rows, cols]
    ids = rows * _N + cols

    out = pl.pallas_call(
        _b_body,
        in_specs=[
            pl.BlockSpec((1, 1), lambda: (0, 0)),
            pl.BlockSpec((_M, 1), lambda: (0, 0)),
            pl.BlockSpec((1, _M), lambda: (0, 0)),
            pl.BlockSpec((_M, 1), lambda: (0, 0)),
            pl.BlockSpec((_M, _NUM_CLASSES), lambda: (0, 0)),
            pl.BlockSpec((_M, 1), lambda: (0, 0)),
            pl.BlockSpec((_M, 4), lambda: (0, 0)),
            pl.BlockSpec((_M, 4), lambda: (0, 0)),
            pl.BlockSpec((_M, 1), lambda: (0, 0)),
        ],
        out_specs=pl.BlockSpec((1, 1), lambda: (0, 0)),
        out_shape=jax.ShapeDtypeStruct((1, 1), f32),
    )(noobj, ids.reshape(_M, 1), ids.reshape(1, _M), ti2,
      sel[:, 5:], sel[:, 4:5], csel, boxes,
      y_cls.reshape(_M, 1))

    return out.reshape(())


# speedup vs baseline: 5.7123x; 1.5674x over previous
import functools

import jax
import jax.numpy as jnp
import numpy as np
from jax import lax
from jax.experimental import pallas as pl
from jax.experimental.pallas import tpu as pltpu

_GRID_SIZES = (19, 38, 76)
_INP_DIM = 608.0
_NUM_ANCH = 3
_L_COORD = 1.0
_L_OBJ = 5.0
_L_NOOBJ = 0.5
_B, _T = 16, 8
_N = 3 * (19 * 19 + 38 * 38 + 76 * 76)
_C = 85
_NUM_CLASSES = 80
_M = _B * _T
_K = 9
_NTILE = -(-_N // 128)

_TN = 2048
_NSTEP = -(-_N // _TN)

_AW = (116., 156., 373., 30., 62., 59., 10., 16., 33.)
_AH = (90., 198., 326., 61., 45., 119., 13., 30., 23.)


def _a_body(confT_ref, noobj_ref):
    j = pl.program_id(0)
    conf = confT_ref[0]
    lane = lax.broadcasted_iota(jnp.int32, (_B, _TN), 1) + j * _TN
    x = jnp.where(lane < _N, 1.0 - conf, 1.0)
    s = jnp.sum(jnp.maximum(jnp.log(x), -100.0))

    @pl.when(j == 0)
    def _init():
        noobj_ref[...] = s.reshape(1, 1)

    @pl.when(j > 0)
    def _acc():
        noobj_ref[...] += s.reshape(1, 1)


def _g1_body(bq_ref, l0_ref, ta_ref, tb_ref, tbox_ref,
             pa_ref, pb_ref, candisT_ref,
             cols_ref, ti_ref, bi_ref, bk_ref):
    del ta_ref, tb_ref
    t = pl.program_id(0)
    m = t // 3
    s = t - 3 * m

    @pl.when(t == 0)
    def _init():
        bi_ref[...] = jnp.full((1, _M), -1.0, jnp.float32)
        bk_ref[...] = jnp.zeros((1, _M), jnp.int32)

    blk = jnp.concatenate([pa_ref[...], pb_ref[...]], axis=2)
    sub = lax.broadcasted_iota(jnp.int32, (1, _B, 256), 1)
    rowv = jnp.sum(jnp.where(sub == bq_ref[t], blk, 0.0), axis=1)
    cx, cy = rowv[0:1, :], rowv[1:2, :]
    cw, ch = rowv[2:3, :], rowv[3:4, :]
    x1, y1 = cx - cw / 2.0, cy - ch / 2.0
    x2, y2 = cx + cw / 2.0, cy + ch / 2.0
    tb0, tb1 = tbox_ref[t, 0], tbox_ref[t, 1]
    tb2, tb3 = tbox_ref[t, 2], tbox_ref[t, 3]
    ix1 = jnp.maximum(tb0, x1)
    iy1 = jnp.maximum(tb1, y1)
    ix2 = jnp.minimum(tb2, x2)
    iy2 = jnp.minimum(tb3, y2)
    inter = jnp.maximum(ix2 - ix1, 0.0) * jnp.maximum(iy2 - iy1, 0.0)
    a1 = (tb2 - tb0) * (tb3 - tb1)
    a2 = (x2 - x1) * (y2 - y1)
    iou = inter / (a1 + a2 - inter + 1e-16)

    l0 = l0_ref[t]
    lane = lax.broadcasted_iota(jnp.int32, (1, 256), 1)
    cmask = (lane >= l0) & (lane < l0 + _NUM_ANCH)
    li = jnp.max(jnp.where(cmask, iou, -1.0))
    lo = jnp.min(jnp.where(cmask & (iou == li), lane - l0, _K))
    kloc = _NUM_ANCH * s + lo

    lane_m = lax.broadcasted_iota(jnp.int32, (1, _M), 1)
    upd = (lane_m == m) & (li > bi_ref[...])
    bi_ref[...] = jnp.where(upd, li, bi_ref[...])
    bk_ref[...] = jnp.where(upd, kloc, bk_ref[...])

    @pl.when(t == 3 * _M - 1)
    def _fin():
        bk = bk_ref[...]
        sub9 = lax.broadcasted_iota(jnp.int32, (_K, _M), 0)
        cols_ref[...] = jnp.sum(
            jnp.where(sub9 == bk, candisT_ref[...], 0), axis=0, keepdims=True)
        ti_ref[...] = bk


def _g2_body(bq8_ref, bmod_ref, tcol_ref, lmod_ref,
             pm_ref, cm_ref, noobj_ref, ids_c_ref, ids_r_ref, tiT_ref,
             boxesT_ref, ycls_ref, out_ref, sel_ref, csel_ref):
    del bq8_ref, tcol_ref
    m = pl.program_id(0)

    @pl.when(m == 0)
    def _init():
        sel_ref[...] = jnp.zeros((_C, 1, 128), jnp.float32)
        csel_ref[...] = jnp.zeros((4, 1, 128), jnp.float32)

    lane3 = lax.broadcasted_iota(jnp.int32, (1, 1, 128), 2)
    sub3 = lax.broadcasted_iota(jnp.int32, (1, 8, 1), 1)
    pick = (lane3 == lmod_ref[m]) & (sub3 == bmod_ref[m])
    v = jnp.sum(jnp.where(pick, pm_ref[...], 0.0), axis=(1, 2),
                keepdims=True)
    cv = jnp.sum(jnp.where(pick, cm_ref[...], 0.0), axis=(1, 2),
                 keepdims=True)
    sel_ref[...] += jnp.where(lane3 == m, v, 0.0)
    csel_ref[...] += jnp.where(lane3 == m, cv, 0.0)

    @pl.when(m == _M - 1)
    def _fin():
        selS = sel_ref[...]
        coordS = csel_ref[...]

        ti = tiT_ref[...]
        gidx = ti // _NUM_ANCH
        gf = jnp.where(gidx == 0, 19.0, jnp.where(gidx == 1, 38.0, 76.0))
        aw = jnp.full((1, _M), _AW[0], jnp.float32)
        ah = jnp.full((1, _M), _AH[0], jnp.float32)
        for k in range(1, _K):
            aw = jnp.where(ti == k, _AW[k], aw)
            ah = jnp.where(ti == k, _AH[k], ah)

        boxesT = boxesT_ref[...]
        bx, by = boxesT[0:1, :], boxesT[1:2, :]
        bw, bh = boxesT[2:3, :], boxesT[3:4, :]
        fx = bx * gf
        fy = by * gf
        fx = fx - jnp.floor(fx) + 1e-05
        fy = fy - jnp.floor(fy) + 1e-05
        tx = jnp.log(fx / (1.0 - fx))
        ty = jnp.log(fy / (1.0 - fy))
        tw = jnp.log(bw * _INP_DIM / aw)
        th = jnp.log(bh * _INP_DIM / ah)
        cs0 = coordS[0:1, 0, :]
        cs1 = coordS[1:2, 0, :]
        cs2 = coordS[2:3, 0, :]
        cs3 = coordS[3:4, 0, :]
        coord_loss = _L_COORD * jnp.sum(
            (cs0 - tx) ** 2 + (cs1 - ty) ** 2
            + (cs2 - tw) ** 2 + (cs3 - th) ** 2)

        p = selS[5:, 0, :]
        c80 = lax.broadcasted_iota(jnp.int32, (_NUM_CLASSES, _M), 0)
        tcls = jnp.where(c80 == ycls_ref[...], 1.0, 0.0)
        cls_loss = -jnp.sum(
            jnp.maximum(jnp.log(p), -100.0) * tcls
            + jnp.maximum(jnp.log(1.0 - p), -100.0) * (1.0 - tcls))

        eq = ids_c_ref[...] == ids_r_ref[...]
        lower = (lax.broadcasted_iota(jnp.int32, (_M, _M), 0)
                 < lax.broadcasted_iota(jnp.int32, (_M, _M), 1))
        ndup = jnp.sum(jnp.where(eq & lower, 1.0, 0.0), axis=0,
                       keepdims=True)
        first = jnp.where(ndup > 0.0, 0.0, 1.0)
        cg = selS[4:5, 0, :]
        logp = jnp.maximum(jnp.log(cg), -100.0)
        l1m = jnp.maximum(jnp.log(1.0 - cg), -100.0)
        noobj_total = jnp.sum(noobj_ref[...])
        conf_loss = (_L_OBJ * (-jnp.sum(logp * first))
                     + _L_NOOBJ * (-(noobj_total - jnp.sum(l1m * first))))
        out_ref[...] = (coord_loss + conf_loss + cls_loss).reshape(1, 1)


def kernel(pred_x, coord_x, y_cls, y_coord):
    f32, i32 = jnp.float32, jnp.int32
    predT = jnp.transpose(pred_x, (2, 0, 1))
    coordT = jnp.transpose(coord_x, (2, 0, 1))

    boxes = y_coord.reshape(-1, 4)
    rows = jnp.repeat(jnp.arange(_B, dtype=i32), _T)
    cand_parts = []
    base = 0
    for g in _GRID_SIZES:
        idx0 = base + ((jnp.floor(boxes[:, 1] * g)
                        + jnp.floor(boxes[:, 0] * g) * g) * _NUM_ANCH
                       ).astype(i32)
        cand_parts.append(idx0[:, None]
                          + jnp.arange(_NUM_ANCH, dtype=i32)[None, :])
        base += g * g * _NUM_ANCH
    candis = jnp.concatenate(cand_parts, axis=1)
    tb = jnp.stack([_INP_DIM * (boxes[:, 0] - boxes[:, 2] / 2),
                    _INP_DIM * (boxes[:, 1] - boxes[:, 3] / 2),
                    _INP_DIM * (boxes[:, 0] + boxes[:, 2] / 2),
                    _INP_DIM * (boxes[:, 1] + boxes[:, 3] / 2)], axis=1)

    noobj = pl.pallas_call(
        _a_body,
        grid=(_NSTEP,),
        in_specs=[pl.BlockSpec((1, _B, _TN), lambda j: (4, 0, j))],
        out_specs=pl.BlockSpec((1, 1), lambda j: (0, 0)),
        out_shape=jax.ShapeDtypeStruct((1, 1), f32),
    )(predT)

    idx0_steps = candis[:, ::_NUM_ANCH].reshape(-1)
    ta = idx0_steps // 128
    tb_tile = jnp.minimum(ta + 1, _NTILE - 1)
    l0 = idx0_steps % 128
    bq384 = jnp.repeat(jnp.arange(_B, dtype=i32), 3 * _T)
    tbox384 = jnp.repeat(tb, 3, axis=0)

    cols_row, ti_row = pl.pallas_call(
        _g1_body,
        grid_spec=pltpu.PrefetchScalarGridSpec(
            num_scalar_prefetch=5,
            grid=(3 * _M,),
            in_specs=[
                pl.BlockSpec((4, _B, 128),
                             lambda t, bq, l0r, tar, tbr, tbox: (0, 0, tar[t])),
                pl.BlockSpec((4, _B, 128),
                             lambda t, bq, l0r, tar, tbr, tbox: (0, 0, tbr[t])),
                pl.BlockSpec((_K, _M),
                             lambda t, bq, l0r, tar, tbr, tbox: (0, 0)),
            ],
            out_specs=[
                pl.BlockSpec((1, _M),
                             lambda t, bq, l0r, tar, tbr, tbox: (0, 0)),
                pl.BlockSpec((1, _M),
                             lambda t, bq, l0r, tar, tbr, tbox: (0, 0)),
            ],
            scratch_shapes=[
                pltpu.VMEM((1, _M), f32),
                pltpu.VMEM((1, _M), i32),
            ],
        ),
        out_shape=[
            jax.ShapeDtypeStruct((1, _M), i32),
            jax.ShapeDtypeStruct((1, _M), i32),
        ],
    )(bq384, l0, ta, tb_tile, tbox384, predT, predT, candis.T)

    cols = cols_row.reshape(-1)
    ids = rows * _N + cols
    tcol = cols // 128
    lmod = cols % 128

    out = pl.pallas_call(
        _g2_body,
        grid_spec=pltpu.PrefetchScalarGridSpec(
            num_scalar_prefetch=4,
            grid=(_M,),
            in_specs=[
                pl.BlockSpec((_C, 8, 128),
                             lambda m, bq8, bm, tc, lm: (0, bq8[m], tc[m])),
                pl.BlockSpec((4, 8, 128),
                             lambda m, bq8, bm, tc, lm: (0, bq8[m], tc[m])),
                pl.BlockSpec((1, 1), lambda m, bq8, bm, tc, lm: (0, 0)),
                pl.BlockSpec((_M, 1), lambda m, bq8, bm, tc, lm: (0, 0)),
                pl.BlockSpec((1, _M), lambda m, bq8, bm, tc, lm: (0, 0)),
                pl.BlockSpec((1, _M), lambda m, bq8, bm, tc, lm: (0, 0)),
                pl.BlockSpec((4, _M), lambda m, bq8, bm, tc, lm: (0, 0)),
                pl.BlockSpec((1, _M), lambda m, bq8, bm, tc, lm: (0, 0)),
            ],
            out_specs=pl.BlockSpec((1, 1), lambda m, bq8, bm, tc, lm: (0, 0)),
            scratch_shapes=[
                pltpu.VMEM((_C, 1, 128), f32),
                pltpu.VMEM((4, 1, 128), f32),
            ],
        ),
        out_shape=jax.ShapeDtypeStruct((1, 1), f32),
    )(rows // 8, rows % 8, tcol, lmod, predT, coordT, noobj,
      ids.reshape(_M, 1), ids.reshape(1, _M), ti_row,
      boxes.T, y_cls.reshape(1, _M))

    return out.reshape(())


# speedup vs baseline: 8.8335x; 1.5464x over previous
import functools

import jax
import jax.numpy as jnp
import numpy as np
from jax import lax
from jax.experimental import pallas as pl
from jax.experimental.pallas import tpu as pltpu

_GRID_SIZES = (19, 38, 76)
_INP_DIM = 608.0
_NUM_ANCH = 3
_L_COORD = 1.0
_L_OBJ = 5.0
_L_NOOBJ = 0.5
_B, _T = 16, 8
_N = 3 * (19 * 19 + 38 * 38 + 76 * 76)
_C = 85
_NUM_CLASSES = 80
_M = _B * _T
_K = 9
_NTILE = -(-_N // 128)

_TN = 2048
_NSTEP = -(-_N // _TN)

_AW = (116., 156., 373., 30., 62., 59., 10., 16., 33.)
_AH = (90., 198., 326., 61., 45., 119., 13., 30., 23.)


def _a_body(confT_ref, noobj_ref):
    j = pl.program_id(0)
    conf = confT_ref[0]
    lane = lax.broadcasted_iota(jnp.int32, (_B, _TN), 1) + j * _TN
    x = jnp.where(lane < _N, 1.0 - conf, 1.0)
    s = jnp.sum(jnp.maximum(jnp.log(x), -100.0))

    @pl.when(j == 0)
    def _init():
        noobj_ref[...] = s.reshape(1, 1)

    @pl.when(j > 0)
    def _acc():
        noobj_ref[...] += s.reshape(1, 1)


def _g1_body(bq8_ref, bmod_ref, l0_ref, ta_ref, tb_ref, tbox_ref,
             pa0_ref, pb0_ref, pa1_ref, pb1_ref, pa2_ref, pb2_ref,
             candisT_ref, cols_ref, ti_ref, bi_ref, bk_ref):
    del bq8_ref, ta_ref, tb_ref
    m = pl.program_id(0)

    @pl.when(m == 0)
    def _init():
        bi_ref[...] = jnp.full((1, _M), -1.0, jnp.float32)
        bk_ref[...] = jnp.zeros((1, _M), jnp.int32)

    sub = lax.broadcasted_iota(jnp.int32, (1, 8, 128), 1)
    bsel = sub == bmod_ref[m]
    lane = lax.broadcasted_iota(jnp.int32, (1, 1, 128), 2)
    tb0, tb1 = tbox_ref[m, 0], tbox_ref[m, 1]
    tb2, tb3 = tbox_ref[m, 2], tbox_ref[m, 3]
    a1 = (tb2 - tb0) * (tb3 - tb1)
    lane_m = lax.broadcasted_iota(jnp.int32, (1, _M), 1)

    def half_iou(blk):
        rowv = jnp.sum(jnp.where(bsel, blk, 0.0), axis=1,
                       keepdims=True)
        cx, cy = rowv[0:1], rowv[1:2]
        cw, ch = rowv[2:3], rowv[3:4]
        x1, y1 = cx - cw / 2.0, cy - ch / 2.0
        x2, y2 = cx + cw / 2.0, cy + ch / 2.0
        ix1 = jnp.maximum(tb0, x1)
        iy1 = jnp.maximum(tb1, y1)
        ix2 = jnp.minimum(tb2, x2)
        iy2 = jnp.minimum(tb3, y2)
        inter = jnp.maximum(ix2 - ix1, 0.0) * jnp.maximum(iy2 - iy1, 0.0)
        a2 = (x2 - x1) * (y2 - y1)
        return inter / (a1 + a2 - inter + 1e-16)

    for s, (pa, pb) in enumerate([(pa0_ref, pb0_ref), (pa1_ref, pb1_ref),
                                  (pa2_ref, pb2_ref)]):
        iou_a = half_iou(pa[...])
        iou_b = half_iou(pb[...])
        l0 = l0_ref[m, s]
        mask_a = (lane >= l0) & (lane < l0 + _NUM_ANCH)
        mask_b = (lane + 128 >= l0) & (lane + 128 < l0 + _NUM_ANCH)
        li = jnp.maximum(jnp.max(jnp.where(mask_a, iou_a, -1.0)),
                         jnp.max(jnp.where(mask_b, iou_b, -1.0)))
        lo = jnp.minimum(
            jnp.min(jnp.where(mask_a & (iou_a == li), lane - l0, _K)),
            jnp.min(jnp.where(mask_b & (iou_b == li), lane + 128 - l0, _K)))
        kloc = _NUM_ANCH * s + lo
        upd = (lane_m == m) & (li > bi_ref[...])
        bi_ref[...] = jnp.where(upd, li, bi_ref[...])
        bk_ref[...] = jnp.where(upd, kloc, bk_ref[...])

    @pl.when(m == _M - 1)
    def _fin():
        bk = bk_ref[...]
        sub9 = lax.broadcasted_iota(jnp.int32, (_K, _M), 0)
        cols_ref[...] = jnp.sum(
            jnp.where(sub9 == bk, candisT_ref[...], 0), axis=0, keepdims=True)
        ti_ref[...] = bk


def _g2_body(bq8_ref, bmod_ref, tcol_ref, lmod_ref,
             pm_ref, cm_ref, noobj_ref, ids_c_ref, ids_r_ref, tiT_ref,
             boxesT_ref, ycls_ref, out_ref, sel_ref, csel_ref):
    del bq8_ref, tcol_ref
    m = pl.program_id(0)

    @pl.when(m == 0)
    def _init():
        sel_ref[...] = jnp.zeros((_C, 1, 128), jnp.float32)
        csel_ref[...] = jnp.zeros((4, 1, 128), jnp.float32)

    lane3 = lax.broadcasted_iota(jnp.int32, (1, 1, 128), 2)
    sub3 = lax.broadcasted_iota(jnp.int32, (1, 8, 1), 1)
    pick = (lane3 == lmod_ref[m]) & (sub3 == bmod_ref[m])
    v = jnp.sum(jnp.where(pick, pm_ref[...], 0.0), axis=(1, 2),
                keepdims=True)
    cv = jnp.sum(jnp.where(pick, cm_ref[...], 0.0), axis=(1, 2),
                 keepdims=True)
    sel_ref[...] += jnp.where(lane3 == m, v, 0.0)
    csel_ref[...] += jnp.where(lane3 == m, cv, 0.0)

    @pl.when(m == _M - 1)
    def _fin():
        selS = sel_ref[...]
        coordS = csel_ref[...]

        ti = tiT_ref[...]
        gidx = ti // _NUM_ANCH
        gf = jnp.where(gidx == 0, 19.0, jnp.where(gidx == 1, 38.0, 76.0))
        aw = jnp.full((1, _M), _AW[0], jnp.float32)
        ah = jnp.full((1, _M), _AH[0], jnp.float32)
        for k in range(1, _K):
            aw = jnp.where(ti == k, _AW[k], aw)
            ah = jnp.where(ti == k, _AH[k], ah)

        boxesT = boxesT_ref[...]
        bx, by = boxesT[0:1, :], boxesT[1:2, :]
        bw, bh = boxesT[2:3, :], boxesT[3:4, :]
        fx = bx * gf
        fy = by * gf
        fx = fx - jnp.floor(fx) + 1e-05
        fy = fy - jnp.floor(fy) + 1e-05
        tx = jnp.log(fx / (1.0 - fx))
        ty = jnp.log(fy / (1.0 - fy))
        tw = jnp.log(bw * _INP_DIM / aw)
        th = jnp.log(bh * _INP_DIM / ah)
        cs0 = coordS[0:1, 0, :]
        cs1 = coordS[1:2, 0, :]
        cs2 = coordS[2:3, 0, :]
        cs3 = coordS[3:4, 0, :]
        coord_loss = _L_COORD * jnp.sum(
            (cs0 - tx) ** 2 + (cs1 - ty) ** 2
            + (cs2 - tw) ** 2 + (cs3 - th) ** 2)

        p = selS[5:, 0, :]
        c80 = lax.broadcasted_iota(jnp.int32, (_NUM_CLASSES, _M), 0)
        tcls = jnp.where(c80 == ycls_ref[...], 1.0, 0.0)
        cls_loss = -jnp.sum(
            jnp.maximum(jnp.log(p), -100.0) * tcls
            + jnp.maximum(jnp.log(1.0 - p), -100.0) * (1.0 - tcls))

        eq = ids_c_ref[...] == ids_r_ref[...]
        lower = (lax.broadcasted_iota(jnp.int32, (_M, _M), 0)
                 < lax.broadcasted_iota(jnp.int32, (_M, _M), 1))
        ndup = jnp.sum(jnp.where(eq & lower, 1.0, 0.0), axis=0,
                       keepdims=True)
        first = jnp.where(ndup > 0.0, 0.0, 1.0)
        cg = selS[4:5, 0, :]
        logp = jnp.maximum(jnp.log(cg), -100.0)
        l1m = jnp.maximum(jnp.log(1.0 - cg), -100.0)
        noobj_total = jnp.sum(noobj_ref[...])
        conf_loss = (_L_OBJ * (-jnp.sum(logp * first))
                     + _L_NOOBJ * (-(noobj_total - jnp.sum(l1m * first))))
        out_ref[...] = (coord_loss + conf_loss + cls_loss).reshape(1, 1)


def kernel(pred_x, coord_x, y_cls, y_coord):
    f32, i32 = jnp.float32, jnp.int32
    predT = jnp.transpose(pred_x, (2, 0, 1))
    coordT = jnp.transpose(coord_x, (2, 0, 1))

    boxes = y_coord.reshape(-1, 4)
    rows = jnp.repeat(jnp.arange(_B, dtype=i32), _T)
    cand_parts = []
    base = 0
    for g in _GRID_SIZES:
        idx0 = base + ((jnp.floor(boxes[:, 1] * g)
                        + jnp.floor(boxes[:, 0] * g) * g) * _NUM_ANCH
                       ).astype(i32)
        cand_parts.append(idx0[:, None]
                          + jnp.arange(_NUM_ANCH, dtype=i32)[None, :])
        base += g * g * _NUM_ANCH
    candis = jnp.concatenate(cand_parts, axis=1)
    tb = jnp.stack([_INP_DIM * (boxes[:, 0] - boxes[:, 2] / 2),
                    _INP_DIM * (boxes[:, 1] - boxes[:, 3] / 2),
                    _INP_DIM * (boxes[:, 0] + boxes[:, 2] / 2),
                    _INP_DIM * (boxes[:, 1] + boxes[:, 3] / 2)], axis=1)

    noobj = pl.pallas_call(
        _a_body,
        grid=(_NSTEP,),
        in_specs=[pl.BlockSpec((1, _B, _TN), lambda j: (4, 0, j))],
        out_specs=pl.BlockSpec((1, 1), lambda j: (0, 0)),
        out_shape=jax.ShapeDtypeStruct((1, 1), f32),
    )(predT)

    idx0 = candis[:, ::_NUM_ANCH]
    ta = idx0 // 128
    tb_tile = jnp.minimum(ta + 1, _NTILE - 1)
    l0 = idx0 % 128

    def _pa(s):
        return pl.BlockSpec(
            (4, 8, 128),
            lambda m, bq8, bm, l0r, tar, tbr, tbox: (0, bq8[m], tar[m, s]))

    def _pb(s):
        return pl.BlockSpec(
            (4, 8, 128),
            lambda m, bq8, bm, l0r, tar, tbr, tbox: (0, bq8[m], tbr[m, s]))

    cols_row, ti_row = pl.pallas_call(
        _g1_body,
        grid_spec=pltpu.PrefetchScalarGridSpec(
            num_scalar_prefetch=6,
            grid=(_M,),
            in_specs=[
                _pa(0), _pb(0), _pa(1), _pb(1), _pa(2), _pb(2),
                pl.BlockSpec((_K, _M),
                             lambda m, bq8, bm, l0r, tar, tbr, tbox: (0, 0)),
            ],
            out_specs=[
                pl.BlockSpec((1, _M),
                             lambda m, bq8, bm, l0r, tar, tbr, tbox: (0, 0)),
                pl.BlockSpec((1, _M),
                             lambda m, bq8, bm, l0r, tar, tbr, tbox: (0, 0)),
            ],
            scratch_shapes=[
                pltpu.VMEM((1, _M), f32),
                pltpu.VMEM((1, _M), i32),
            ],
        ),
        out_shape=[
            jax.ShapeDtypeStruct((1, _M), i32),
            jax.ShapeDtypeStruct((1, _M), i32),
        ],
    )(rows // 8, rows % 8, l0, ta, tb_tile, tb,
      predT, predT, predT, predT, predT, predT, candis.T)

    cols = cols_row.reshape(-1)
    ids = rows * _N + cols
    tcol = cols // 128
    lmod = cols % 128

    out = pl.pallas_call(
        _g2_body,
        grid_spec=pltpu.PrefetchScalarGridSpec(
            num_scalar_prefetch=4,
            grid=(_M,),
            in_specs=[
                pl.BlockSpec((_C, 8, 128),
                             lambda m, bq8, bm, tc, lm: (0, bq8[m], tc[m])),
                pl.BlockSpec((4, 8, 128),
                             lambda m, bq8, bm, tc, lm: (0, bq8[m], tc[m])),
                pl.BlockSpec((1, 1), lambda m, bq8, bm, tc, lm: (0, 0)),
                pl.BlockSpec((_M, 1), lambda m, bq8, bm, tc, lm: (0, 0)),
                pl.BlockSpec((1, _M), lambda m, bq8, bm, tc, lm: (0, 0)),
                pl.BlockSpec((1, _M), lambda m, bq8, bm, tc, lm: (0, 0)),
                pl.BlockSpec((4, _M), lambda m, bq8, bm, tc, lm: (0, 0)),
                pl.BlockSpec((1, _M), lambda m, bq8, bm, tc, lm: (0, 0)),
            ],
            out_specs=pl.BlockSpec((1, 1), lambda m, bq8, bm, tc, lm: (0, 0)),
            scratch_shapes=[
                pltpu.VMEM((_C, 1, 128), f32),
                pltpu.VMEM((4, 1, 128), f32),
            ],
        ),
        out_shape=jax.ShapeDtypeStruct((1, 1), f32),
    )(rows // 8, rows % 8, tcol, lmod, predT, coordT, noobj,
      ids.reshape(_M, 1), ids.reshape(1, _M), ti_row,
      boxes.T, y_cls.reshape(1, _M))

    return out.reshape(())


# speedup vs baseline: 9.6968x; 1.0977x over previous
import functools

import jax
import jax.numpy as jnp
import numpy as np
from jax import lax
from jax.experimental import pallas as pl
from jax.experimental.pallas import tpu as pltpu

_GRID_SIZES = (19, 38, 76)
_INP_DIM = 608.0
_NUM_ANCH = 3
_L_COORD = 1.0
_L_OBJ = 5.0
_L_NOOBJ = 0.5
_B, _T = 16, 8
_N = 3 * (19 * 19 + 38 * 38 + 76 * 76)
_C = 85
_NUM_CLASSES = 80
_M = _B * _T
_K = 9
_NTILE = -(-_N // 128)

_TN = 2048
_NSTEP = -(-_N // _TN)

_AW = (116., 156., 373., 30., 62., 59., 10., 16., 33.)
_AH = (90., 198., 326., 61., 45., 119., 13., 30., 23.)


def _a_body(confT_ref, noobj_ref):
    j = pl.program_id(0)
    conf = confT_ref[0]
    lane = lax.broadcasted_iota(jnp.int32, (_B, _TN), 1) + j * _TN
    x = jnp.where(lane < _N, 1.0 - conf, 1.0)
    s = jnp.sum(jnp.maximum(jnp.log(x), -100.0))

    @pl.when(j == 0)
    def _init():
        noobj_ref[...] = s.reshape(1, 1)

    @pl.when(j > 0)
    def _acc():
        noobj_ref[...] += s.reshape(1, 1)


def _g1_body(bq8_ref, bmod_ref, l0_ref, ta_ref, tb_ref, tbox_ref,
             pa0_ref, pb0_ref, pa1_ref, pb1_ref, pa2_ref, pb2_ref,
             candis_ref, tbv_ref, cols_ref, ti_ref,
             cx_ref, cy_ref, cw_ref, ch_ref):
    del bq8_ref
    m = pl.program_id(0)

    @pl.when(m == 0)
    def _init():
        cx_ref[...] = jnp.zeros((_M, _K), jnp.float32)
        cy_ref[...] = jnp.zeros((_M, _K), jnp.float32)
        cw_ref[...] = jnp.zeros((_M, _K), jnp.float32)
        ch_ref[...] = jnp.zeros((_M, _K), jnp.float32)

    bm = bmod_ref[m]
    r32 = lax.broadcasted_iota(jnp.int32, (4, 32), 1)
    c32 = lax.broadcasted_iota(jnp.int32, (4, 32), 0)
    esel = jnp.where(r32 == 8 * c32 + bm, 1.0, 0.0)

    lane9 = lax.broadcasted_iota(jnp.int32, (128, _K), 0)
    k9 = lax.broadcasted_iota(jnp.int32, (128, _K), 1)
    lane128 = lax.broadcasted_iota(jnp.int32, (1, 128), 1)

    acc = jnp.zeros((4, _K), jnp.float32)
    for s, (pa, pb) in enumerate([(pa0_ref, pb0_ref), (pa1_ref, pb1_ref),
                                  (pa2_ref, pb2_ref)]):
        l0 = l0_ref[m, s]
        band = (k9 >= 3 * s) & (k9 < 3 * s + 3)
        oneh_a = jnp.where(band & (lane9 == l0 + k9 - 3 * s), 1.0, 0.0)
        oneh_b = jnp.where(band & (lane9 + 128 == l0 + k9 - 3 * s), 1.0, 0.0)
        rowa = jnp.dot(esel, pa[...].reshape(32, 128),
                       preferred_element_type=jnp.float32)
        rowb = jnp.dot(esel, pb[...].reshape(32, 128),
                       preferred_element_type=jnp.float32)
        rowa = jnp.where(ta_ref[m, s] * 128 + lane128 < _N, rowa, 0.0)
        rowb = jnp.where(tb_ref[m, s] * 128 + lane128 < _N, rowb, 0.0)
        acc = acc + jnp.dot(rowa, oneh_a, preferred_element_type=jnp.float32)
        acc = acc + jnp.dot(rowb, oneh_b, preferred_element_type=jnp.float32)

    rowm = lax.broadcasted_iota(jnp.int32, (_M, _K), 0) == m
    cx_ref[...] += jnp.where(rowm, acc[0:1, :], 0.0)
    cy_ref[...] += jnp.where(rowm, acc[1:2, :], 0.0)
    cw_ref[...] += jnp.where(rowm, acc[2:3, :], 0.0)
    ch_ref[...] += jnp.where(rowm, acc[3:4, :], 0.0)

    @pl.when(m == _M - 1)
    def _fin():
        cx, cy = cx_ref[...], cy_ref[...]
        cw, ch = cw_ref[...], ch_ref[...]
        x1, y1 = cx - cw / 2.0, cy - ch / 2.0
        x2, y2 = cx + cw / 2.0, cy + ch / 2.0
        tbv = tbv_ref[...]
        ix1 = jnp.maximum(tbv[:, 0:1], x1)
        iy1 = jnp.maximum(tbv[:, 1:2], y1)
        ix2 = jnp.minimum(tbv[:, 2:3], x2)
        iy2 = jnp.minimum(tbv[:, 3:4], y2)
        inter = jnp.maximum(ix2 - ix1, 0.0) * jnp.maximum(iy2 - iy1, 0.0)
        a1 = (tbv[:, 2:3] - tbv[:, 0:1]) * (tbv[:, 3:4] - tbv[:, 1:2])
        a2 = (x2 - x1) * (y2 - y1)
        iou = inter / (a1 + a2 - inter + 1e-16)
        kio = lax.broadcasted_iota(jnp.int32, (_M, _K), 1)
        mx = jnp.max(iou, axis=1, keepdims=True)
        ti = jnp.min(jnp.where(iou == mx, kio, _K), axis=1, keepdims=True)
        cols_ref[...] = jnp.sum(jnp.where(kio == ti, candis_ref[...], 0),
                                axis=1, keepdims=True)
        ti_ref[...] = ti


def _g2_body(bq8_ref, bmod_ref, tcol_ref, lmod_ref,
             pm_ref, cm_ref, noobj_ref, ids_c_ref, ids_r_ref, tiT_ref,
             boxesT_ref, ycls_ref, out_ref, sel_ref, csel_ref):
    del bq8_ref, tcol_ref
    m = pl.program_id(0)

    @pl.when(m == 0)
    def _init():
        sel_ref[...] = jnp.zeros((_C, 1, 128), jnp.float32)
        csel_ref[...] = jnp.zeros((4, 1, 128), jnp.float32)

    lane3 = lax.broadcasted_iota(jnp.int32, (1, 1, 128), 2)
    sub3 = lax.broadcasted_iota(jnp.int32, (1, 8, 1), 1)
    pick = (lane3 == lmod_ref[m]) & (sub3 == bmod_ref[m])
    v = jnp.sum(jnp.where(pick, pm_ref[...], 0.0), axis=(1, 2),
                keepdims=True)
    cv = jnp.sum(jnp.where(pick, cm_ref[...], 0.0), axis=(1, 2),
                 keepdims=True)
    sel_ref[...] += jnp.where(lane3 == m, v, 0.0)
    csel_ref[...] += jnp.where(lane3 == m, cv, 0.0)

    @pl.when(m == _M - 1)
    def _fin():
        selS = sel_ref[...]
        coordS = csel_ref[...]

        ti = tiT_ref[...]
        gidx = ti // _NUM_ANCH
        gf = jnp.where(gidx == 0, 19.0, jnp.where(gidx == 1, 38.0, 76.0))
        aw = jnp.full((1, _M), _AW[0], jnp.float32)
        ah = jnp.full((1, _M), _AH[0], jnp.float32)
        for k in range(1, _K):
            aw = jnp.where(ti == k, _AW[k], aw)
            ah = jnp.where(ti == k, _AH[k], ah)

        boxesT = boxesT_ref[...]
        bx, by = boxesT[0:1, :], boxesT[1:2, :]
        bw, bh = boxesT[2:3, :], boxesT[3:4, :]
        fx = bx * gf
        fy = by * gf
        fx = fx - jnp.floor(fx) + 1e-05
        fy = fy - jnp.floor(fy) + 1e-05
        tx = jnp.log(fx / (1.0 - fx))
        ty = jnp.log(fy / (1.0 - fy))
        tw = jnp.log(bw * _INP_DIM / aw)
        th = jnp.log(bh * _INP_DIM / ah)
        cs0 = coordS[0:1, 0, :]
        cs1 = coordS[1:2, 0, :]
        cs2 = coordS[2:3, 0, :]
        cs3 = coordS[3:4, 0, :]
        coord_loss = _L_COORD * jnp.sum(
            (cs0 - tx) ** 2 + (cs1 - ty) ** 2
            + (cs2 - tw) ** 2 + (cs3 - th) ** 2)

        p = selS[5:, 0, :]
        c80 = lax.broadcasted_iota(jnp.int32, (_NUM_CLASSES, _M), 0)
        tcls = jnp.where(c80 == ycls_ref[...], 1.0, 0.0)
        cls_loss = -jnp.sum(
            jnp.maximum(jnp.log(p), -100.0) * tcls
            + jnp.maximum(jnp.log(1.0 - p), -100.0) * (1.0 - tcls))

        eq = ids_c_ref[...] == ids_r_ref[...]
        lower = (lax.broadcasted_iota(jnp.int32, (_M, _M), 0)
                 < lax.broadcasted_iota(jnp.int32, (_M, _M), 1))
        ndup = jnp.sum(jnp.where(eq & lower, 1.0, 0.0), axis=0,
                       keepdims=True)
        first = jnp.where(ndup > 0.0, 0.0, 1.0)
        cg = selS[4:5, 0, :]
        logp = jnp.maximum(jnp.log(cg), -100.0)
        l1m = jnp.maximum(jnp.log(1.0 - cg), -100.0)
        noobj_total = jnp.sum(noobj_ref[...])
        conf_loss = (_L_OBJ * (-jnp.sum(logp * first))
                     + _L_NOOBJ * (-(noobj_total - jnp.sum(l1m * first))))
        out_ref[...] = (coord_loss + conf_loss + cls_loss).reshape(1, 1)


def kernel(pred_x, coord_x, y_cls, y_coord):
    f32, i32 = jnp.float32, jnp.int32
    predT = jnp.transpose(pred_x, (2, 0, 1))
    coordT = jnp.transpose(coord_x, (2, 0, 1))

    boxes = y_coord.reshape(-1, 4)
    rows = jnp.repeat(jnp.arange(_B, dtype=i32), _T)
    cand_parts = []
    base = 0
    for g in _GRID_SIZES:
        idx0 = base + ((jnp.floor(boxes[:, 1] * g)
                        + jnp.floor(boxes[:, 0] * g) * g) * _NUM_ANCH
                       ).astype(i32)
        cand_parts.append(idx0[:, None]
                          + jnp.arange(_NUM_ANCH, dtype=i32)[None, :])
        base += g * g * _NUM_ANCH
    candis = jnp.concatenate(cand_parts, axis=1)
    tb = jnp.stack([_INP_DIM * (boxes[:, 0] - boxes[:, 2] / 2),
                    _INP_DIM * (boxes[:, 1] - boxes[:, 3] / 2),
                    _INP_DIM * (boxes[:, 0] + boxes[:, 2] / 2),
                    _INP_DIM * (boxes[:, 1] + boxes[:, 3] / 2)], axis=1)

    noobj = pl.pallas_call(
        _a_body,
        grid=(_NSTEP,),
        in_specs=[pl.BlockSpec((1, _B, _TN), lambda j: (4, 0, j))],
        out_specs=pl.BlockSpec((1, 1), lambda j: (0, 0)),
        out_shape=jax.ShapeDtypeStruct((1, 1), f32),
    )(predT)

    idx0 = candis[:, ::_NUM_ANCH]
    ta = idx0 // 128
    tb_tile = jnp.minimum(ta + 1, _NTILE - 1)
    l0 = idx0 % 128

    def _pa(s):
        return pl.BlockSpec(
            (4, 8, 128),
            lambda m, bq8, bm, l0r, tar, tbr, tbox: (0, bq8[m], tar[m, s]))

    def _pb(s):
        return pl.BlockSpec(
            (4, 8, 128),
            lambda m, bq8, bm, l0r, tar, tbr, tbox: (0, bq8[m], tbr[m, s]))

    cols2, ti2 = pl.pallas_call(
        _g1_body,
        grid_spec=pltpu.PrefetchScalarGridSpec(
            num_scalar_prefetch=6,
            grid=(_M,),
            in_specs=[
                _pa(0), _pb(0), _pa(1), _pb(1), _pa(2), _pb(2),
                pl.BlockSpec((_M, _K),
                             lambda m, bq8, bm, l0r, tar, tbr, tbox: (0, 0)),
                pl.BlockSpec((_M, 4),
                             lambda m, bq8, bm, l0r, tar, tbr, tbox: (0, 0)),
            ],
            out_specs=[
                pl.BlockSpec((_M, 1),
                             lambda m, bq8, bm, l0r, tar, tbr, tbox: (0, 0)),
                pl.BlockSpec((_M, 1),
                             lambda m, bq8, bm, l0r, tar, tbr, tbox: (0, 0)),
            ],
            scratch_shapes=[
                pltpu.VMEM((_M, _K), f32),
                pltpu.VMEM((_M, _K), f32),
                pltpu.VMEM((_M, _K), f32),
                pltpu.VMEM((_M, _K), f32),
            ],
        ),
        out_shape=[
            jax.ShapeDtypeStruct((_M, 1), i32),
            jax.ShapeDtypeStruct((_M, 1), i32),
        ],
    )(rows // 8, rows % 8, l0, ta, tb_tile, tb,
      predT, predT, predT, predT, predT, predT, candis, tb)

    ti_row = ti2.reshape(1, _M)
    cols = cols2.reshape(-1)
    ids = rows * _N + cols
    tcol = cols // 128
    lmod = cols % 128

    out = pl.pallas_call(
        _g2_body,
        grid_spec=pltpu.PrefetchScalarGridSpec(
            num_scalar_prefetch=4,
            grid=(_M,),
            in_specs=[
                pl.BlockSpec((_C, 8, 128),
                             lambda m, bq8, bm, tc, lm: (0, bq8[m], tc[m])),
                pl.BlockSpec((4, 8, 128),
                             lambda m, bq8, bm, tc, lm: (0, bq8[m], tc[m])),
                pl.BlockSpec((1, 1), lambda m, bq8, bm, tc, lm: (0, 0)),
                pl.BlockSpec((_M, 1), lambda m, bq8, bm, tc, lm: (0, 0)),
                pl.BlockSpec((1, _M), lambda m, bq8, bm, tc, lm: (0, 0)),
                pl.BlockSpec((1, _M), lambda m, bq8, bm, tc, lm: (0, 0)),
                pl.BlockSpec((4, _M), lambda m, bq8, bm, tc, lm: (0, 0)),
                pl.BlockSpec((1, _M), lambda m, bq8, bm, tc, lm: (0, 0)),
            ],
            out_specs=pl.BlockSpec((1, 1), lambda m, bq8, bm, tc, lm: (0, 0)),
            scratch_shapes=[
                pltpu.VMEM((_C, 1, 128), f32),
                pltpu.VMEM((4, 1, 128), f32),
            ],
        ),
        out_shape=jax.ShapeDtypeStruct((1, 1), f32),
    )(rows // 8, rows % 8, tcol, lmod, predT, coordT, noobj,
      ids.reshape(_M, 1), ids.reshape(1, _M), ti_row,
      boxes.T, y_cls.reshape(1, _M))

    return out.reshape(())


# speedup vs baseline: 29.7031x; 3.0632x over previous
import functools

import jax
import jax.numpy as jnp
import numpy as np
from jax import lax
from jax.experimental import pallas as pl

_GRID_SIZES = (19, 38, 76)
_INP_DIM = 608.0
_NUM_ANCH = 3
_L_COORD = 1.0
_L_OBJ = 5.0
_L_NOOBJ = 0.5
_B, _T = 16, 8
_N = 3 * (19 * 19 + 38 * 38 + 76 * 76)
_C = 85
_NUM_CLASSES = 80
_M = _B * _T
_K = 9

_TN = 2048
_NSTEP = -(-_N // _TN)

_AW = (116., 156., 373., 30., 62., 59., 10., 16., 33.)
_AH = (90., 198., 326., 61., 45., 119., 13., 30., 23.)


def _a_body(confT_ref, cx_ref, cy_ref, cw_ref, ch_ref, candis_ref, tb_ref,
            noobj_ref, cols_ref, ti_ref):
    j = pl.program_id(0)
    conf = confT_ref[0]
    lane = lax.broadcasted_iota(jnp.int32, (_B, _TN), 1) + j * _TN
    x = jnp.where(lane < _N, 1.0 - conf, 1.0)
    s = jnp.sum(jnp.maximum(jnp.log(x), -100.0))

    @pl.when(j == 0)
    def _init():
        noobj_ref[...] = s.reshape(1, 1)
        cx, cy = cx_ref[...], cy_ref[...]
        cw, ch = cw_ref[...], ch_ref[...]
        x1, y1 = cx - cw / 2.0, cy - ch / 2.0
        x2, y2 = cx + cw / 2.0, cy + ch / 2.0
        tb = tb_ref[...]
        ix1 = jnp.maximum(tb[:, 0:1], x1)
        iy1 = jnp.maximum(tb[:, 1:2], y1)
        ix2 = jnp.minimum(tb[:, 2:3], x2)
        iy2 = jnp.minimum(tb[:, 3:4], y2)
        inter = jnp.maximum(ix2 - ix1, 0.0) * jnp.maximum(iy2 - iy1, 0.0)
        a1 = (tb[:, 2:3] - tb[:, 0:1]) * (tb[:, 3:4] - tb[:, 1:2])
        a2 = (x2 - x1) * (y2 - y1)
        iou = inter / (a1 + a2 - inter + 1e-16)
        kio = lax.broadcasted_iota(jnp.int32, (_M, _K), 1)
        mx = jnp.max(iou, axis=1, keepdims=True)
        ti = jnp.min(jnp.where(iou == mx, kio, _K), axis=1, keepdims=True)
        cols_ref[...] = jnp.sum(
            jnp.where(kio == ti, candis_ref[...], 0), axis=1, keepdims=True)
        ti_ref[...] = ti

    @pl.when(j > 0)
    def _acc():
        noobj_ref[...] += s.reshape(1, 1)


def _b_body(noobj_ref, ids_c_ref, ids_r_ref, ti_ref, cls_ref, conf_ref,
            csel_ref, boxes_ref, ycls_ref, out_ref):
    ti = ti_ref[...]
    gidx = ti // _NUM_ANCH
    gf = jnp.where(gidx == 0, 19.0, jnp.where(gidx == 1, 38.0, 76.0))
    aw = jnp.full((_M, 1), _AW[0], jnp.float32)
    ah = jnp.full((_M, 1), _AH[0], jnp.float32)
    for k in range(1, _K):
        aw = jnp.where(ti == k, _AW[k], aw)
        ah = jnp.where(ti == k, _AH[k], ah)
    boxes = boxes_ref[...]
    bx, by = boxes[:, 0:1], boxes[:, 1:2]
    bw, bh = boxes[:, 2:3], boxes[:, 3:4]
    fx = bx * gf
    fy = by * gf
    fx = fx - jnp.floor(fx) + 1e-05
    fy = fy - jnp.floor(fy) + 1e-05
    tx = jnp.log(fx / (1.0 - fx))
    ty = jnp.log(fy / (1.0 - fy))
    tw = jnp.log(bw * _INP_DIM / aw)
    th = jnp.log(bh * _INP_DIM / ah)
    cs = csel_ref[...]
    coord_loss = _L_COORD * jnp.sum(
        (cs[:, 0:1] - tx) ** 2 + (cs[:, 1:2] - ty) ** 2
        + (cs[:, 2:3] - tw) ** 2 + (cs[:, 3:4] - th) ** 2)

    c80 = lax.broadcasted_iota(jnp.int32, (_M, _NUM_CLASSES), 1)
    tcls = jnp.where(c80 == ycls_ref[...], 1.0, 0.0)
    p = cls_ref[...]
    cls_loss = -jnp.sum(
        jnp.maximum(jnp.log(p), -100.0) * tcls
        + jnp.maximum(jnp.log(1.0 - p), -100.0) * (1.0 - tcls))

    eq = ids_c_ref[...] == ids_r_ref[...]
    lower = (lax.broadcasted_iota(jnp.int32, (_M, _M), 1)
             < lax.broadcasted_iota(jnp.int32, (_M, _M), 0))
    ndup = jnp.sum(jnp.where(eq & lower, 1.0, 0.0), axis=1, keepdims=True)
    first = jnp.where(ndup > 0.0, 0.0, 1.0)
    cg = conf_ref[...]
    logp = jnp.maximum(jnp.log(cg), -100.0)
    l1m = jnp.maximum(jnp.log(1.0 - cg), -100.0)
    noobj_total = jnp.sum(noobj_ref[...])
    conf_loss = (_L_OBJ * (-jnp.sum(logp * first))
                 + _L_NOOBJ * (-(noobj_total - jnp.sum(l1m * first))))
    out_ref[...] = (coord_loss + conf_loss + cls_loss).reshape(1, 1)


def kernel(pred_x, coord_x, y_cls, y_coord):
    f32, i32 = jnp.float32, jnp.int32
    predT = jnp.transpose(pred_x, (2, 0, 1))
    coordT = jnp.transpose(coord_x, (2, 0, 1))

    boxes = y_coord.reshape(-1, 4)
    rows = jnp.repeat(jnp.arange(_B, dtype=i32), _T)
    cand_parts = []
    base = 0
    for g in _GRID_SIZES:
        idx0 = base + ((jnp.floor(boxes[:, 1] * g)
                        + jnp.floor(boxes[:, 0] * g) * g) * _NUM_ANCH
                       ).astype(i32)
        cand_parts.append(idx0[:, None]
                          + jnp.arange(_NUM_ANCH, dtype=i32)[None, :])
        base += g * g * _NUM_ANCH
    candis = jnp.concatenate(cand_parts, axis=1)
    tb = jnp.stack([_INP_DIM * (boxes[:, 0] - boxes[:, 2] / 2),
                    _INP_DIM * (boxes[:, 1] - boxes[:, 3] / 2),
                    _INP_DIM * (boxes[:, 0] + boxes[:, 2] / 2),
                    _INP_DIM * (boxes[:, 1] + boxes[:, 3] / 2)], axis=1)

    ci_cand = jnp.broadcast_to(
        candis.reshape(1, _B, _T * _K), (_C, _B, _T * _K))
    gat = jnp.take_along_axis(predT, ci_cand, axis=2)[:4]
    cxc = gat[0].reshape(_M, _K)
    cyc = gat[1].reshape(_M, _K)
    cwc = gat[2].reshape(_M, _K)
    chc = gat[3].reshape(_M, _K)

    noobj, cols2, ti2 = pl.pallas_call(
        _a_body,
        grid=(_NSTEP,),
        in_specs=[
            pl.BlockSpec((1, _B, _TN), lambda j: (4, 0, j)),
            pl.BlockSpec((_M, _K), lambda j: (0, 0)),
            pl.BlockSpec((_M, _K), lambda j: (0, 0)),
            pl.BlockSpec((_M, _K), lambda j: (0, 0)),
            pl.BlockSpec((_M, _K), lambda j: (0, 0)),
            pl.BlockSpec((_M, _K), lambda j: (0, 0)),
            pl.BlockSpec((_M, 4), lambda j: (0, 0)),
        ],
        out_specs=[
            pl.BlockSpec((1, 1), lambda j: (0, 0)),
            pl.BlockSpec((_M, 1), lambda j: (0, 0)),
            pl.BlockSpec((_M, 1), lambda j: (0, 0)),
        ],
        out_shape=[
            jax.ShapeDtypeStruct((1, 1), f32),
            jax.ShapeDtypeStruct((_M, 1), i32),
            jax.ShapeDtypeStruct((_M, 1), i32),
        ],
    )(predT, cxc, cyc, cwc, chc, candis, tb)

    cols = cols2.reshape(-1)
    ids = rows * _N + cols

    ci_sel = jnp.broadcast_to(cols.reshape(1, _B, _T), (_C, _B, _T))
    selT = jnp.take_along_axis(predT, ci_sel, axis=2)
    sel = selT.transpose(1, 2, 0).reshape(_M, _C)
    ci_co = jnp.broadcast_to(cols.reshape(1, _B, _T), (4, _B, _T))
    csel = jnp.take_along_axis(coordT, ci_co, axis=2
                               ).transpose(1, 2, 0).reshape(_M, 4)

    out = pl.pallas_call(
        _b_body,
        in_specs=[
            pl.BlockSpec((1, 1), lambda: (0, 0)),
            pl.BlockSpec((_M, 1), lambda: (0, 0)),
            pl.BlockSpec((1, _M), lambda: (0, 0)),
            pl.BlockSpec((_M, 1), lambda: (0, 0)),
            pl.BlockSpec((_M, _NUM_CLASSES), lambda: (0, 0)),
            pl.BlockSpec((_M, 1), lambda: (0, 0)),
            pl.BlockSpec((_M, 4), lambda: (0, 0)),
            pl.BlockSpec((_M, 4), lambda: (0, 0)),
            pl.BlockSpec((_M, 1), lambda: (0, 0)),
        ],
        out_specs=pl.BlockSpec((1, 1), lambda: (0, 0)),
        out_shape=jax.ShapeDtypeStruct((1, 1), f32),
    )(noobj, ids.reshape(_M, 1), ids.reshape(1, _M), ti2,
      sel[:, 5:], sel[:, 4:5], csel, boxes,
      y_cls.reshape(_M, 1))

    return out.reshape(())
